# baseline (speedup 1.0000x reference)
import numpy as np
import jax
import jax.numpy as jnp
from jax.experimental import pallas as pl

F = np.float32
_C4 = F(0.919999957)
_C6 = F(0.920000076)
_BW = F(0.01)
_CP4 = F(F(F(_BW + _BW) + _BW) + _BW)
_TB = F(0.02)
_T0 = F(0.0 - _TB)
_T1 = F(_BW - _TB)
_T2 = F(F(_BW + _BW) - _TB)
_T3 = F(F(F(_BW + _BW) + _BW) - _TB)
_T4 = F(_CP4 - _TB)
_Q2 = F(_BW + _BW)
_A0 = F(0.0 - _BW)
_A1 = F(_BW - _BW)
_A2 = F(_Q2 - _BW)


def _spline_block(x, c0, c1, c2, c3, g0, g1, g2, g3, g4, g5):
    mx = jnp.maximum(jnp.maximum(c0, c1), jnp.maximum(c2, c3))
    E0 = jnp.exp(c0 - mx); E1 = jnp.exp(c1 - mx)
    E2 = jnp.exp(c2 - mx); E3 = jnp.exp(c3 - mx)
    S = (E0 + E2) + (E1 + E3)
    d1 = (E0 / S) * _C4 + _BW
    d2 = (E1 / S) * _C4 + _BW
    d3 = (E2 / S) * _C4 + _BW
    d4 = (E3 / S) * _C4 + _BW

    p5 = _CP4 + d1
    p6 = p5 + d2
    p7 = p6 + d3
    p8 = p7 + d4
    p9 = p8 + _BW
    p10 = p9 + _BW
    p11 = p10 + _BW
    p12 = p11 + _BW
    T5 = p5 - _TB
    T6 = p6 - _TB
    T7 = p7 - _TB
    T8 = p8 - _TB
    T9 = p9 - _TB
    T10 = p10 - _TB
    T11 = p11 - _TB
    T12 = p12 - _TB
    T0 = jnp.full_like(x, _T0); T1 = jnp.full_like(x, _T1)
    T2 = jnp.full_like(x, _T2); T3c = jnp.full_like(x, _T3)
    T4 = jnp.full_like(x, _T4)

    mg = jnp.maximum(jnp.maximum(jnp.maximum(g0, g1), jnp.maximum(g2, g3)),
                     jnp.maximum(g4, g5))
    F0 = jnp.exp(g0 - mg); F1 = jnp.exp(g1 - mg); F2 = jnp.exp(g2 - mg)
    F3 = jnp.exp(g3 - mg); F4 = jnp.exp(g4 - mg); F5 = jnp.exp(g5 - mg)
    SG = ((F0 + F4) + F2) + ((F1 + F5) + F3)
    e1 = (F0 / SG) * _C6 + _BW
    e2 = (F1 / SG) * _C6 + _BW
    e3 = (F2 / SG) * _C6 + _BW
    e4 = (F3 / SG) * _C6 + _BW
    e5 = (F4 / SG) * _C6 + _BW
    e6 = (F5 / SG) * _C6 + _BW

    q3 = _Q2 + e1
    q4 = q3 + e2
    q5 = q4 + e3
    q6 = q5 + e4
    q7 = q6 + e5
    q8 = q7 + e6
    q9 = q8 + _BW
    q10 = q9 + _BW
    A3 = q3 - _BW
    A4 = q4 - _BW
    A5 = q5 - _BW
    A6 = q6 - _BW
    A7 = q7 - _BW
    A8 = q8 - _BW
    A9 = q9 - _BW
    A10 = q10 - _BW
    A0 = jnp.full_like(x, _A0); A1 = jnp.full_like(x, _A1)
    A2 = jnp.full_like(x, _A2)

    m1 = x >= T3c
    m2 = x >= T4
    m3 = x >= T5
    m4 = x >= T6
    m5 = x >= T7
    m6 = x >= T8
    m7 = x >= T9

    def sel(v0, v1, v2, v3, v4, v5, v6, v7):
        x01 = jnp.where(m1, v1, v0)
        x23 = jnp.where(m3, v3, v2)
        x45 = jnp.where(m5, v5, v4)
        x67 = jnp.where(m7, v7, v6)
        y0 = jnp.where(m2, x23, x01)
        y1 = jnp.where(m6, x67, x45)
        return jnp.where(m4, y1, y0)

    tm2 = sel(T0, T1, T2, T3c, T4, T5, T6, T7)
    tm1 = sel(T1, T2, T3c, T4, T5, T6, T7, T8)
    t0 = sel(T2, T3c, T4, T5, T6, T7, T8, T9)
    t1 = sel(T3c, T4, T5, T6, T7, T8, T9, T10)
    t2 = sel(T4, T5, T6, T7, T8, T9, T10, T11)
    t3 = sel(T5, T6, T7, T8, T9, T10, T11, T12)
    km3 = sel(A0, A1, A2, A3, A4, A5, A6, A7)
    km2 = sel(A1, A2, A3, A4, A5, A6, A7, A8)
    km1 = sel(A2, A3, A4, A5, A6, A7, A8, A9)
    km0 = sel(A3, A4, A5, A6, A7, A8, A9, A10)

    one = F(1.0)
    a = km0 * (one / ((t3 - t0) * (t2 - t0) * (t1 - t0))) + km1 * (-one / ((t2 - tm1) * (t1 - tm1) * (t1 - t0)) - one / ((t2 - tm1) * (t2 - t0) * (t1 - t0)) - one / ((t3 - t0) * (t2 - t0) * (t1 - t0))) + km2 * (one / ((t1 - t0) * (t1 - tm2) * (t1 - tm1)) + one / ((t1 - t0) * (t2 - t0) * (t2 - tm1)) + one / ((t1 - t0) * (t1 - tm1) * (t2 - tm1))) + km3 * (-one / ((t1 - tm2) * (t1 - tm1) * (t1 - t0)))
    b = km0 * (F(-3) * t0 / ((t3 - t0) * (t2 - t0) * (t1 - t0))) + km1 * ((F(2) * tm1 + t1) / ((t2 - tm1) * (t1 - tm1) * (t1 - t0)) + (tm1 + t2 + t0) / ((t2 - tm1) * (t2 - t0) * (t1 - t0)) + (t3 + F(2) * t0) / ((t3 - t0) * (t2 - t0) * (t1 - t0))) + km2 * ((-F(2) * t1 - tm2) / ((t1 - t0) * (t1 - tm2) * (t1 - tm1)) + (-F(2) * t2 - t0) / ((t1 - t0) * (t2 - t0) * (t2 - tm1)) + (-t2 - t1 - tm1) / ((t1 - t0) * (t1 - tm1) * (t2 - tm1))) + km3 * (F(3) * t1 / ((t1 - tm2) * (t1 - tm1) * (t1 - t0)))
    c = km0 * (F(3) * t0 * t0 / ((t3 - t0) * (t2 - t0) * (t1 - t0))) + km1 * ((-tm1 * tm1 - F(2) * tm1 * t1) / ((t2 - tm1) * (t1 - tm1) * (t1 - t0)) + (-tm1 * t2 - tm1 * t0 - t2 * t0) / ((t2 - tm1) * (t2 - t0) * (t1 - t0)) + (-t0 * t0 - F(2) * t3 * t0) / ((t3 - t0) * (t2 - t0) * (t1 - t0))) + km2 * ((t1 * t1 + F(2) * t1 * tm2) / ((t1 - t0) * (t1 - tm2) * (t1 - tm1)) + ((t2 + F(2) * t0) * t2) / ((t1 - t0) * (t2 - t0) * (t2 - tm1)) + ((tm1 + t2) * t1 + t2 * tm1) / ((t1 - t0) * (t1 - tm1) * (t2 - tm1))) + km3 * (-F(3) * t1 * t1 / ((t1 - tm2) * (t1 - tm1) * (t1 - t0)))
    d = km0 * (-t0 * t0 * t0 / ((t3 - t0) * (t2 - t0) * (t1 - t0))) + km1 * (tm1 * tm1 * t1 / ((t2 - tm1) * (t1 - tm1) * (t1 - t0)) + tm1 * t2 * t0 / ((t2 - tm1) * (t2 - t0) * (t1 - t0)) + t3 * t0 * t0 / ((t3 - t0) * (t2 - t0) * (t1 - t0))) + km2 * (-(t1 * t1 * tm2) / ((t1 - t0) * (t1 - tm2) * (t1 - tm1)) - t0 * t2 * t2 / ((t1 - t0) * (t2 - t0) * (t2 - tm1)) - t2 * tm1 * t1 / ((t1 - t0) * (t1 - tm1) * (t2 - tm1))) + km3 * (t1 * t1 * t1 / ((t1 - tm2) * (t1 - tm1) * (t1 - t0)))

    out_in = ((a * x + b) * x + c) * x + d
    deriv = (F(3.0) * a * x + F(2.0) * b) * x + c
    lad_in = jnp.log(jnp.abs(deriv) + F(1e-12))

    inside = jnp.logical_and(x > F(0.0), x < F(1.0))
    outputs = jnp.where(inside, out_in, x)
    logabsdet = jnp.where(inside, lad_in, F(0.0))
    return outputs, logabsdet


def _kernel_body(x_ref, dt_ref, da_ref, out_ref, lad_ref):
    x = x_ref[...]
    outputs, logabsdet = _spline_block(
        x,
        dt_ref[0], dt_ref[1], dt_ref[2], dt_ref[3],
        da_ref[0], da_ref[1], da_ref[2], da_ref[3], da_ref[4], da_ref[5],
    )
    out_ref[...] = outputs
    lad_ref[...] = logabsdet


_BR = 512


_NCHUNK = 2


def _run_chunk(x2, dt3, da3):
    nr = x2.shape[0]
    grid = (nr // _BR,)
    return pl.pallas_call(
        _kernel_body,
        grid=grid,
        in_specs=[
            pl.BlockSpec((_BR, 128), lambda i: (i, 0)),
            pl.BlockSpec((4, _BR, 128), lambda i: (0, i, 0)),
            pl.BlockSpec((6, _BR, 128), lambda i: (0, i, 0)),
        ],
        out_specs=[
            pl.BlockSpec((_BR, 128), lambda i: (i, 0)),
            pl.BlockSpec((_BR, 128), lambda i: (i, 0)),
        ],
        out_shape=[
            jax.ShapeDtypeStruct((nr, 128), jnp.float32),
            jax.ShapeDtypeStruct((nr, 128), jnp.float32),
        ],
    )(x2, dt3, da3)


def kernel(inputs, unnormalized_dt, unnormalized_dalpha):
    n = inputs.shape[0]
    nr = n // 128
    step = nr // _NCHUNK
    outs, lads = [], []
    for c in range(_NCHUNK):
        sl = slice(c * step * 128, (c + 1) * step * 128)
        o, l = _run_chunk(
            inputs[sl].reshape(step, 128),
            unnormalized_dt[sl].T.reshape(4, step, 128),
            unnormalized_dalpha[sl].T.reshape(6, step, 128),
        )
        outs.append(o.reshape(step * 128))
        lads.append(l.reshape(step * 128))
    return jnp.concatenate(outs), jnp.concatenate(lads)


# speedup vs baseline: 1.3675x; 1.3675x over previous
import numpy as np
import jax
import jax.numpy as jnp
from jax.experimental import pallas as pl

F = np.float32
_C4 = F(0.919999957)
_C6 = F(0.920000076)
_BW = F(0.01)
_CP4 = F(F(F(_BW + _BW) + _BW) + _BW)
_TB = F(0.02)
_T0 = F(0.0 - _TB)
_T1 = F(_BW - _TB)
_T2 = F(F(_BW + _BW) - _TB)
_T3 = F(F(F(_BW + _BW) + _BW) - _TB)
_T4 = F(_CP4 - _TB)
_Q2 = F(_BW + _BW)
_A0 = F(0.0 - _BW)
_A1 = F(_BW - _BW)
_A2 = F(_Q2 - _BW)


def _spline_block(x, c0, c1, c2, c3, g0, g1, g2, g3, g4, g5):
    mx = jnp.maximum(jnp.maximum(c0, c1), jnp.maximum(c2, c3))
    E0 = jnp.exp(c0 - mx); E1 = jnp.exp(c1 - mx)
    E2 = jnp.exp(c2 - mx); E3 = jnp.exp(c3 - mx)
    S = (E0 + E2) + (E1 + E3)
    d1 = (E0 / S) * _C4 + _BW
    d2 = (E1 / S) * _C4 + _BW
    d3 = (E2 / S) * _C4 + _BW
    d4 = (E3 / S) * _C4 + _BW

    p5 = _CP4 + d1
    p6 = p5 + d2
    p7 = p6 + d3
    p8 = p7 + d4
    p9 = p8 + _BW
    p10 = p9 + _BW
    p11 = p10 + _BW
    p12 = p11 + _BW
    T5 = p5 - _TB
    T6 = p6 - _TB
    T7 = p7 - _TB
    T8 = p8 - _TB
    T9 = p9 - _TB
    T10 = p10 - _TB
    T11 = p11 - _TB
    T12 = p12 - _TB
    T0 = jnp.full_like(x, _T0); T1 = jnp.full_like(x, _T1)
    T2 = jnp.full_like(x, _T2); T3c = jnp.full_like(x, _T3)
    T4 = jnp.full_like(x, _T4)

    mg = jnp.maximum(jnp.maximum(jnp.maximum(g0, g1), jnp.maximum(g2, g3)),
                     jnp.maximum(g4, g5))
    F0 = jnp.exp(g0 - mg); F1 = jnp.exp(g1 - mg); F2 = jnp.exp(g2 - mg)
    F3 = jnp.exp(g3 - mg); F4 = jnp.exp(g4 - mg); F5 = jnp.exp(g5 - mg)
    SG = ((F0 + F4) + F2) + ((F1 + F5) + F3)
    e1 = (F0 / SG) * _C6 + _BW
    e2 = (F1 / SG) * _C6 + _BW
    e3 = (F2 / SG) * _C6 + _BW
    e4 = (F3 / SG) * _C6 + _BW
    e5 = (F4 / SG) * _C6 + _BW
    e6 = (F5 / SG) * _C6 + _BW

    q3 = _Q2 + e1
    q4 = q3 + e2
    q5 = q4 + e3
    q6 = q5 + e4
    q7 = q6 + e5
    q8 = q7 + e6
    q9 = q8 + _BW
    q10 = q9 + _BW
    A3 = q3 - _BW
    A4 = q4 - _BW
    A5 = q5 - _BW
    A6 = q6 - _BW
    A7 = q7 - _BW
    A8 = q8 - _BW
    A9 = q9 - _BW
    A10 = q10 - _BW
    A0 = jnp.full_like(x, _A0); A1 = jnp.full_like(x, _A1)
    A2 = jnp.full_like(x, _A2)

    m1 = x >= T3c
    m2 = x >= T4
    m3 = x >= T5
    m4 = x >= T6
    m5 = x >= T7
    m6 = x >= T8
    m7 = x >= T9

    def sel(v0, v1, v2, v3, v4, v5, v6, v7):
        x01 = jnp.where(m1, v1, v0)
        x23 = jnp.where(m3, v3, v2)
        x45 = jnp.where(m5, v5, v4)
        x67 = jnp.where(m7, v7, v6)
        y0 = jnp.where(m2, x23, x01)
        y1 = jnp.where(m6, x67, x45)
        return jnp.where(m4, y1, y0)

    tm2 = sel(T0, T1, T2, T3c, T4, T5, T6, T7)
    tm1 = sel(T1, T2, T3c, T4, T5, T6, T7, T8)
    t0 = sel(T2, T3c, T4, T5, T6, T7, T8, T9)
    t1 = sel(T3c, T4, T5, T6, T7, T8, T9, T10)
    t2 = sel(T4, T5, T6, T7, T8, T9, T10, T11)
    t3 = sel(T5, T6, T7, T8, T9, T10, T11, T12)
    km3 = sel(A0, A1, A2, A3, A4, A5, A6, A7)
    km2 = sel(A1, A2, A3, A4, A5, A6, A7, A8)
    km1 = sel(A2, A3, A4, A5, A6, A7, A8, A9)
    km0 = sel(A3, A4, A5, A6, A7, A8, A9, A10)

    one = F(1.0)
    a = km0 * (one / ((t3 - t0) * (t2 - t0) * (t1 - t0))) + km1 * (-one / ((t2 - tm1) * (t1 - tm1) * (t1 - t0)) - one / ((t2 - tm1) * (t2 - t0) * (t1 - t0)) - one / ((t3 - t0) * (t2 - t0) * (t1 - t0))) + km2 * (one / ((t1 - t0) * (t1 - tm2) * (t1 - tm1)) + one / ((t1 - t0) * (t2 - t0) * (t2 - tm1)) + one / ((t1 - t0) * (t1 - tm1) * (t2 - tm1))) + km3 * (-one / ((t1 - tm2) * (t1 - tm1) * (t1 - t0)))
    b = km0 * (F(-3) * t0 / ((t3 - t0) * (t2 - t0) * (t1 - t0))) + km1 * ((F(2) * tm1 + t1) / ((t2 - tm1) * (t1 - tm1) * (t1 - t0)) + (tm1 + t2 + t0) / ((t2 - tm1) * (t2 - t0) * (t1 - t0)) + (t3 + F(2) * t0) / ((t3 - t0) * (t2 - t0) * (t1 - t0))) + km2 * ((-F(2) * t1 - tm2) / ((t1 - t0) * (t1 - tm2) * (t1 - tm1)) + (-F(2) * t2 - t0) / ((t1 - t0) * (t2 - t0) * (t2 - tm1)) + (-t2 - t1 - tm1) / ((t1 - t0) * (t1 - tm1) * (t2 - tm1))) + km3 * (F(3) * t1 / ((t1 - tm2) * (t1 - tm1) * (t1 - t0)))
    c = km0 * (F(3) * t0 * t0 / ((t3 - t0) * (t2 - t0) * (t1 - t0))) + km1 * ((-tm1 * tm1 - F(2) * tm1 * t1) / ((t2 - tm1) * (t1 - tm1) * (t1 - t0)) + (-tm1 * t2 - tm1 * t0 - t2 * t0) / ((t2 - tm1) * (t2 - t0) * (t1 - t0)) + (-t0 * t0 - F(2) * t3 * t0) / ((t3 - t0) * (t2 - t0) * (t1 - t0))) + km2 * ((t1 * t1 + F(2) * t1 * tm2) / ((t1 - t0) * (t1 - tm2) * (t1 - tm1)) + ((t2 + F(2) * t0) * t2) / ((t1 - t0) * (t2 - t0) * (t2 - tm1)) + ((tm1 + t2) * t1 + t2 * tm1) / ((t1 - t0) * (t1 - tm1) * (t2 - tm1))) + km3 * (-F(3) * t1 * t1 / ((t1 - tm2) * (t1 - tm1) * (t1 - t0)))
    d = km0 * (-t0 * t0 * t0 / ((t3 - t0) * (t2 - t0) * (t1 - t0))) + km1 * (tm1 * tm1 * t1 / ((t2 - tm1) * (t1 - tm1) * (t1 - t0)) + tm1 * t2 * t0 / ((t2 - tm1) * (t2 - t0) * (t1 - t0)) + t3 * t0 * t0 / ((t3 - t0) * (t2 - t0) * (t1 - t0))) + km2 * (-(t1 * t1 * tm2) / ((t1 - t0) * (t1 - tm2) * (t1 - tm1)) - t0 * t2 * t2 / ((t1 - t0) * (t2 - t0) * (t2 - tm1)) - t2 * tm1 * t1 / ((t1 - t0) * (t1 - tm1) * (t2 - tm1))) + km3 * (t1 * t1 * t1 / ((t1 - tm2) * (t1 - tm1) * (t1 - t0)))

    out_in = ((a * x + b) * x + c) * x + d
    deriv = (F(3.0) * a * x + F(2.0) * b) * x + c
    lad_in = jnp.log(jnp.abs(deriv) + F(1e-12))

    inside = jnp.logical_and(x > F(0.0), x < F(1.0))
    outputs = jnp.where(inside, out_in, x)
    logabsdet = jnp.where(inside, lad_in, F(0.0))
    return outputs, logabsdet


def _kernel_body(x_ref, dt_ref, da_ref, out_ref, lad_ref):
    x = x_ref[...]
    outputs, logabsdet = _spline_block(
        x,
        dt_ref[0], dt_ref[1], dt_ref[2], dt_ref[3],
        da_ref[0], da_ref[1], da_ref[2], da_ref[3], da_ref[4], da_ref[5],
    )
    out_ref[...] = outputs
    lad_ref[...] = logabsdet


_BR = 1024


def kernel(inputs, unnormalized_dt, unnormalized_dalpha):
    n = inputs.shape[0]
    nr = n // 128
    x2 = inputs.reshape(nr, 128)
    dt3 = unnormalized_dt.T.reshape(4, nr, 128)
    da3 = unnormalized_dalpha.T.reshape(6, nr, 128)
    grid = (nr // _BR,)
    out, lad = pl.pallas_call(
        _kernel_body,
        grid=grid,
        in_specs=[
            pl.BlockSpec((_BR, 128), lambda i: (i, 0)),
            pl.BlockSpec((4, _BR, 128), lambda i: (0, i, 0)),
            pl.BlockSpec((6, _BR, 128), lambda i: (0, i, 0)),
        ],
        out_specs=[
            pl.BlockSpec((_BR, 128), lambda i: (i, 0)),
            pl.BlockSpec((_BR, 128), lambda i: (i, 0)),
        ],
        out_shape=[
            jax.ShapeDtypeStruct((nr, 128), jnp.float32),
            jax.ShapeDtypeStruct((nr, 128), jnp.float32),
        ],
    )(x2, dt3, da3)
    return out.reshape(n), lad.reshape(n)
